# baseline (speedup 1.0000x reference)
import dataclasses

import jax
import jax.numpy as jnp
from jax import lax
from jax.experimental import pallas as pl
from jax.experimental.pallas import tpu as pltpu
from jax.experimental.pallas import tpu_sc as plsc

_NW = 32


def _sc_agg(B, T, D, L, K):
    SPW = (K * L) // _NW
    NCH = SPW // 16
    mesh = plsc.VectorSubcoreMesh(core_axis_name="c", subcore_axis_name="s")

    def body(xflat_hbm, spans_hbm, out_hbm, spans_v, idx_v, rows_v, agg_v, sem):
        wid = lax.axis_index("s") * 2 + lax.axis_index("c")
        lane = lax.iota(jnp.int32, 16)
        r_lane = lane & 7
        p_hi = lane >> 3

        for ch in range(NCH):
            sp_flat = wid * SPW + ch * 16
            bi_k = sp_flat // L
            j0 = sp_flat - bi_k * L
            bi = (B - K) + bi_k
            pltpu.sync_copy(spans_hbm.at[bi, pl.ds(j0, 16)], spans_v)

            for g in range(8):
                p_vec = g * 2 + p_hi
                zero = jnp.zeros((16,), jnp.int32)
                one = jnp.ones((16,), jnp.int32)
                s_vec = plsc.load_gather(spans_v, [p_vec, zero])
                e_vec = plsc.load_gather(spans_v, [p_vec, one])
                w_vec = e_vec - s_vec
                ridx = bi * T + s_vec + jnp.minimum(r_lane, w_vec - 1)
                idx_v[pl.ds(g * 16, 16)] = ridx

            pltpu.async_copy(xflat_hbm.at[idx_v], rows_v, sem).wait()

            s16 = plsc.load_gather(spans_v, [lane, jnp.zeros((16,), jnp.int32)])
            e16 = plsc.load_gather(spans_v, [lane, jnp.ones((16,), jnp.int32)])
            w16 = (e16 - s16).astype(jnp.float32)

            @pl.loop(0, 16)
            def _(p):
                wb = jnp.take(w16, jnp.full((16,), p, jnp.int32))
                inv_w = 1.0 / wb
                corr = 8.0 - wb
                for c in range(D // 16):
                    sl = pl.ds(c * 16, 16)
                    acc = rows_v[8 * p + 0, sl]
                    for r in range(1, 7):
                        acc = acc + rows_v[8 * p + r, sl]
                    last = rows_v[8 * p + 7, sl]
                    acc = acc + last
                    agg_v[p, sl] = (acc - corr * last) * inv_w

            pltpu.sync_copy(agg_v, out_hbm.at[bi_k, pl.ds(j0, 16)])

    cp = pltpu.CompilerParams()
    if "needs_layout_passes" in pltpu.CompilerParams.__dataclass_fields__:
        cp = dataclasses.replace(cp, needs_layout_passes=False)
    return pl.kernel(
        body,
        out_type=jax.ShapeDtypeStruct((K, L, D), jnp.float32),
        mesh=mesh,
        compiler_params=cp,
        scratch_types=[
            pltpu.VMEM((16, 2), jnp.int32),
            pltpu.VMEM((128,), jnp.int32),
            pltpu.VMEM((128, D), jnp.float32),
            pltpu.VMEM((16, D), jnp.float32),
            pltpu.SemaphoreType.DMA,
        ],
    )


def _linear_kernel(len_ref, agg_ref, W_ref, b_ref, out_ref):
    L = agg_ref.shape[1]
    j_iota = jax.lax.broadcasted_iota(jnp.int32, (L, 1), 0)
    valid = (j_iota < len_ref[pl.program_id(0)]).astype(jnp.float32)
    agg = agg_ref[0] * valid
    out_ref[0] = (
        jnp.dot(agg.astype(jnp.bfloat16), W_ref[...].T.astype(jnp.bfloat16),
                preferred_element_type=jnp.float32)
        + b_ref[...]
    )


def kernel(input, lengths, span_indexes, W, b):
    B, T, D = input.shape
    L = span_indexes.shape[1]
    xflat = input.reshape(B * T, D)

    sc_agg = _sc_agg(B, T, D, L, B)(xflat, span_indexes)

    out = pl.pallas_call(
        _linear_kernel,
        grid=(B,),
        in_specs=[
            pl.BlockSpec(memory_space=pltpu.SMEM),
            pl.BlockSpec((1, L, D), lambda i: (i, 0, 0)),
            pl.BlockSpec((D, D), lambda i: (0, 0)),
            pl.BlockSpec((1, D), lambda i: (0, 0)),
        ],
        out_specs=pl.BlockSpec((1, L, D), lambda i: (i, 0, 0)),
        out_shape=jax.ShapeDtypeStruct((B, L, D), jnp.float32),
    )(lengths, sc_agg, W, b.reshape(1, D))
    return out


# speedup vs baseline: 2.3610x; 2.3610x over previous
import dataclasses

import jax
import jax.numpy as jnp
from jax import lax
from jax.experimental import pallas as pl
from jax.experimental.pallas import tpu as pltpu
from jax.experimental.pallas import tpu_sc as plsc

_NW = 32
_K = 1
_NCHUNK = 2


def _sc_agg(B, T, D, L, K):
    SPW = (K * L) // _NW
    CS = min(16, SPW)
    NCH = SPW // CS
    mesh = plsc.VectorSubcoreMesh(core_axis_name="c", subcore_axis_name="s")

    def body(xflat_hbm, spans_hbm, out_hbm, spans_v, idx_v, rows_v, agg_v, sem):
        wid = lax.axis_index("s") * 2 + lax.axis_index("c")
        lane = lax.iota(jnp.int32, 16)
        r_lane = lane & 7
        p_hi = lane >> 3

        for ch in range(NCH):
            sp_flat = wid * SPW + ch * CS
            bi_k = sp_flat // L
            j0 = sp_flat - bi_k * L
            bi = (B - K) + bi_k
            pltpu.sync_copy(spans_hbm.at[bi, pl.ds(j0, CS)], spans_v)

            for g in range(CS // 2):
                p_vec = g * 2 + p_hi
                zero = jnp.zeros((16,), jnp.int32)
                one = jnp.ones((16,), jnp.int32)
                s_vec = plsc.load_gather(spans_v, [p_vec, zero])
                e_vec = plsc.load_gather(spans_v, [p_vec, one])
                w_vec = e_vec - s_vec
                ridx = bi * T + s_vec + jnp.minimum(r_lane, w_vec - 1)
                idx_v[pl.ds(g * 16, 16)] = ridx

            pltpu.async_copy(xflat_hbm.at[idx_v], rows_v, sem).wait()

            lane_c = lane & (CS - 1)
            s16 = plsc.load_gather(spans_v, [lane_c, jnp.zeros((16,), jnp.int32)])
            e16 = plsc.load_gather(spans_v, [lane_c, jnp.ones((16,), jnp.int32)])
            w16 = (e16 - s16).astype(jnp.float32)

            @pl.loop(0, CS)
            def _(p):
                wb = jnp.take(w16, jnp.full((16,), p, jnp.int32))
                inv_w = 1.0 / wb
                corr = 8.0 - wb
                for c in range(D // 16):
                    sl = pl.ds(c * 16, 16)
                    acc = rows_v[8 * p + 0, sl]
                    for r in range(1, 7):
                        acc = acc + rows_v[8 * p + r, sl]
                    last = rows_v[8 * p + 7, sl]
                    acc = acc + last
                    agg_v[p, sl] = (acc - corr * last) * inv_w

            pltpu.sync_copy(agg_v, out_hbm.at[bi_k, pl.ds(j0, CS)])

    cp = pltpu.CompilerParams()
    if "needs_layout_passes" in pltpu.CompilerParams.__dataclass_fields__:
        cp = dataclasses.replace(cp, needs_layout_passes=False)
    return pl.kernel(
        body,
        out_type=jax.ShapeDtypeStruct((K, L, D), jnp.float32),
        mesh=mesh,
        compiler_params=cp,
        scratch_types=[
            pltpu.VMEM((CS, 2), jnp.int32),
            pltpu.VMEM((CS * 8,), jnp.int32),
            pltpu.VMEM((CS * 8, D), jnp.float32),
            pltpu.VMEM((CS, D), jnp.float32),
            pltpu.SemaphoreType.DMA,
        ],
    )


def _tc_main(x_hbm, len_ref, spans_ref, W_ref, b_ref, out_hbm,
             xbuf, obuf, in_sems, out_sems):
    BT, T, D = x_hbm.shape[0] - _K, x_hbm.shape[1], x_hbm.shape[2]
    L = spans_ref.shape[1]
    Tc = T // _NCHUNK

    for ci in range(BT * _NCHUNK):
        bi, c = divmod(ci, _NCHUNK)
        pltpu.make_async_copy(
            x_hbm.at[bi, pl.ds(c * Tc, Tc)],
            xbuf.at[bi, pl.ds(c * Tc, Tc)],
            in_sems.at[ci],
        ).start()

    Wt = W_ref[...].T.astype(jnp.bfloat16)
    bias = b_ref[...]

    for bi in range(BT):
        ii = spans_ref[bi, :, 0:1]
        jj = spans_ref[bi, :, 1:2]
        width = (jj - ii).astype(jnp.float32)
        j_iota = jax.lax.broadcasted_iota(jnp.int32, (L, 1), 0)
        valid = (j_iota < len_ref[bi]).astype(jnp.float32)
        scale = valid / width
        agg = None
        for c in range(_NCHUNK):
            pltpu.make_async_copy(
                x_hbm.at[bi, pl.ds(c * Tc, Tc)],
                xbuf.at[bi, pl.ds(c * Tc, Tc)],
                in_sems.at[bi * _NCHUNK + c],
            ).wait()
            t = c * Tc + jax.lax.broadcasted_iota(jnp.int32, (L, Tc), 1)
            mask = (t >= ii) & (t < jj)
            M = jnp.where(mask, scale, 0.0)
            part = jnp.dot(
                M.astype(jnp.bfloat16),
                xbuf[bi, c * Tc:(c + 1) * Tc].astype(jnp.bfloat16),
                preferred_element_type=jnp.float32,
            )
            agg = part if agg is None else agg + part
        obuf[bi] = (
            jnp.dot(agg.astype(jnp.bfloat16), Wt,
                    preferred_element_type=jnp.float32)
            + bias
        )
        pltpu.make_async_copy(obuf.at[bi], out_hbm.at[bi],
                              out_sems.at[bi]).start()

    for bi in range(BT):
        pltpu.make_async_copy(obuf.at[bi], out_hbm.at[bi],
                              out_sems.at[bi]).wait()


def _tc_fixup(len_ref, main_ref, agg_ref, W_ref, b_ref, out_ref):
    L = agg_ref.shape[1]
    B = len_ref.shape[0]
    j_iota = jax.lax.broadcasted_iota(jnp.int32, (L, 1), 0)
    bi = B - _K + pl.program_id(0)
    valid = (j_iota < len_ref[bi]).astype(jnp.float32)
    agg = agg_ref[0] * valid
    out_ref[0] = (
        jnp.dot(agg.astype(jnp.bfloat16), W_ref[...].T.astype(jnp.bfloat16),
                preferred_element_type=jnp.float32)
        + b_ref[...]
    )


def kernel(input, lengths, span_indexes, W, b):
    B, T, D = input.shape
    L = span_indexes.shape[1]
    b2 = b.reshape(1, D)
    xflat = input.reshape(B * T, D)

    sc_agg = _sc_agg(B, T, D, L, _K)(xflat, span_indexes)

    main = pl.pallas_call(
        _tc_main,
        in_specs=[
            pl.BlockSpec(memory_space=pltpu.MemorySpace.HBM),
            pl.BlockSpec(memory_space=pltpu.SMEM),
            pl.BlockSpec((B, L, 2), lambda: (0, 0, 0)),
            pl.BlockSpec((D, D), lambda: (0, 0)),
            pl.BlockSpec((1, D), lambda: (0, 0)),
        ],
        out_specs=pl.BlockSpec(memory_space=pltpu.MemorySpace.HBM),
        out_shape=jax.ShapeDtypeStruct((B, L, D), jnp.float32),
        scratch_shapes=[
            pltpu.VMEM((B - _K, T, D), jnp.float32),
            pltpu.VMEM((B - _K, L, D), jnp.float32),
            pltpu.SemaphoreType.DMA(((B - _K) * _NCHUNK,)),
            pltpu.SemaphoreType.DMA((B - _K,)),
        ],
    )(input, lengths, span_indexes, W, b2)

    out = pl.pallas_call(
        _tc_fixup,
        grid=(_K,),
        in_specs=[
            pl.BlockSpec(memory_space=pltpu.SMEM),
            pl.BlockSpec(memory_space=pltpu.MemorySpace.HBM),
            pl.BlockSpec((1, L, D), lambda i: (i, 0, 0)),
            pl.BlockSpec((D, D), lambda i: (0, 0)),
            pl.BlockSpec((1, D), lambda i: (0, 0)),
        ],
        out_specs=pl.BlockSpec((1, L, D), lambda i: (B - _K + i, 0, 0)),
        out_shape=jax.ShapeDtypeStruct((B, L, D), jnp.float32),
        input_output_aliases={1: 0},
    )(lengths, main, sc_agg, W, b2)
    return out


# speedup vs baseline: 4.9317x; 2.0888x over previous
import jax
import jax.numpy as jnp
from jax.experimental import pallas as pl
from jax.experimental.pallas import tpu as pltpu

_NCHUNK = 2


def _agg_kernel(x_hbm, len_ref, spans_ref, W_ref, b_ref, out_hbm,
                xbuf, obuf, in_sems, out_sems):
    B, T, D = x_hbm.shape
    L = spans_ref.shape[1]
    Tc = T // _NCHUNK

    for ci in range(B * _NCHUNK):
        bi, c = divmod(ci, _NCHUNK)
        pltpu.make_async_copy(
            x_hbm.at[bi, pl.ds(c * Tc, Tc)],
            xbuf.at[bi, pl.ds(c * Tc, Tc)],
            in_sems.at[ci],
        ).start()

    Wt = W_ref[...].T.astype(jnp.bfloat16)
    bias = b_ref[...]

    for bi in range(B):
        ii = spans_ref[bi, :, 0:1]
        jj = spans_ref[bi, :, 1:2]
        width = (jj - ii).astype(jnp.float32)
        j_iota = jax.lax.broadcasted_iota(jnp.int32, (L, 1), 0)
        valid = (j_iota < len_ref[bi]).astype(jnp.float32)
        scale = valid / width
        agg = None
        for c in range(_NCHUNK):
            pltpu.make_async_copy(
                x_hbm.at[bi, pl.ds(c * Tc, Tc)],
                xbuf.at[bi, pl.ds(c * Tc, Tc)],
                in_sems.at[bi * _NCHUNK + c],
            ).wait()
            t = c * Tc + jax.lax.broadcasted_iota(jnp.int32, (L, Tc), 1)
            mask = (t >= ii) & (t < jj)
            M = jnp.where(mask, scale, 0.0)
            part = jnp.dot(
                M.astype(jnp.bfloat16),
                xbuf[bi, c * Tc:(c + 1) * Tc].astype(jnp.bfloat16),
                preferred_element_type=jnp.float32,
            )
            agg = part if agg is None else agg + part
        obuf[bi] = (
            jnp.dot(agg.astype(jnp.bfloat16), Wt,
                    preferred_element_type=jnp.float32)
            + bias
        )
        pltpu.make_async_copy(obuf.at[bi], out_hbm.at[bi],
                              out_sems.at[bi]).start()

    for bi in range(B):
        pltpu.make_async_copy(obuf.at[bi], out_hbm.at[bi],
                              out_sems.at[bi]).wait()


def kernel(input, lengths, span_indexes, W, b):
    B, T, D = input.shape
    L = span_indexes.shape[1]

    out = pl.pallas_call(
        _agg_kernel,
        in_specs=[
            pl.BlockSpec(memory_space=pltpu.MemorySpace.HBM),
            pl.BlockSpec(memory_space=pltpu.SMEM),
            pl.BlockSpec((B, L, 2), lambda: (0, 0, 0)),
            pl.BlockSpec((D, D), lambda: (0, 0)),
            pl.BlockSpec((1, D), lambda: (0, 0)),
        ],
        out_specs=pl.BlockSpec(memory_space=pltpu.MemorySpace.HBM),
        out_shape=jax.ShapeDtypeStruct((B, L, D), jnp.float32),
        scratch_shapes=[
            pltpu.VMEM((B, T, D), jnp.float32),
            pltpu.VMEM((B, L, D), jnp.float32),
            pltpu.SemaphoreType.DMA((B * _NCHUNK,)),
            pltpu.SemaphoreType.DMA((B,)),
        ],
    )(input, lengths, span_indexes, W, b.reshape(1, D))
    return out
